# baseline (speedup 1.0000x reference)
import jax
import jax.numpy as jnp
import numpy as np
from jax.experimental import pallas as pl
from jax.experimental.pallas import tpu as pltpu

_INV_EM1 = float(1.0 / (np.exp(1.0) - 1.0))
_MU = 3.5
_G = 25
_R = 1250
_L = 128


def _blend_block(p, v):
    lane = jax.lax.broadcasted_iota(jnp.int32, p.shape, 1)
    is_x = (lane & 1) == 0
    p_l = pltpu.roll(p, _L - 1, 1)
    p_r = pltpu.roll(p, 1, 1)
    x = jnp.where(is_x, p, p_r)
    y = jnp.where(is_x, p_l, p)
    m = (x >= 0.0) & (x <= 0.25) & (y >= 0.0) & (y <= 1.0)
    xr = jnp.clip(x * 4.0, 0.0, 1.0)
    t = jnp.exp(jnp.log(xr) * _MU)
    g = (jnp.exp(t) - 1.0) * _INV_EM1
    g = jnp.minimum(g, 1.0)
    cs = jnp.where(is_x, 1.0, 0.0)
    v_out = jnp.where(m, v + g * (cs - v), v)
    return v_out, g


def _vel_kernel(pos_ref, vel_ref, velout_ref, gamma_ref):
    p = pos_ref[0]
    v = vel_ref[0]
    v_out, g = _blend_block(p, v)
    velout_ref[0] = v_out
    rows = jax.lax.broadcasted_iota(jnp.int32, (_L, _L // 2), 0)
    cols = jax.lax.broadcasted_iota(jnp.int32, (_L, _L // 2), 1)
    sel = (rows == 2 * cols).astype(jnp.float32)
    gamma_ref[0] = jax.lax.dot(g, sel, preferred_element_type=jnp.float32)


def kernel(fluidPosition, fluidVelocity, fluidArea):
    n = fluidPosition.shape[0]
    pos = fluidPosition.reshape(_G, _R, _L)
    vel = fluidVelocity.reshape(_G, _R, _L)
    vel_out, gamma = pl.pallas_call(
        _vel_kernel,
        grid=(_G,),
        in_specs=[
            pl.BlockSpec((1, _R, _L), lambda i: (i, 0, 0)),
            pl.BlockSpec((1, _R, _L), lambda i: (i, 0, 0)),
        ],
        out_specs=[
            pl.BlockSpec((1, _R, _L), lambda i: (i, 0, 0)),
            pl.BlockSpec((1, _R, _L // 2), lambda i: (i, 0, 0)),
        ],
        out_shape=[
            jax.ShapeDtypeStruct((_G, _R, _L), jnp.float32),
            jax.ShapeDtypeStruct((_G, _R, _L // 2), jnp.float32),
        ],
    )(pos, vel)
    return vel_out.reshape(n, 2), gamma.reshape(n)


# speedup vs baseline: 1.1913x; 1.1913x over previous
import jax
import jax.numpy as jnp
import numpy as np
from jax.experimental import pallas as pl
from jax.experimental.pallas import tpu as pltpu

_INV_EM1 = float(1.0 / (np.exp(1.0) - 1.0))
_MU = 3.5
_L = 128
_R = 1024
_B = _R * _L


def _blend_block(p, v):
    lane = jax.lax.broadcasted_iota(jnp.int32, p.shape, 1)
    is_x = (lane & 1) == 0
    p_l = pltpu.roll(p, _L - 1, 1)
    p_r = pltpu.roll(p, 1, 1)
    x = jnp.where(is_x, p, p_r)
    y = jnp.where(is_x, p_l, p)
    m = (x >= 0.0) & (x <= 0.25) & (y >= 0.0) & (y <= 1.0)
    xr = jnp.clip(x * 4.0, 0.0, 1.0)
    t = jnp.exp(jnp.log(xr) * _MU)
    g = (jnp.exp(t) - 1.0) * _INV_EM1
    g = jnp.minimum(g, 1.0)
    cs = jnp.where(is_x, 1.0, 0.0)
    v_out = jnp.where(m, v + g * (cs - v), v)
    return v_out, g


def _vel_kernel(pos_ref, vel_ref, velout_ref, gamma_ref):
    p = pos_ref[...].reshape(_R, _L)
    v = vel_ref[...].reshape(_R, _L)
    v_out, g = _blend_block(p, v)
    velout_ref[...] = v_out.reshape(_B)
    rows = jax.lax.broadcasted_iota(jnp.int32, (_L, _L), 0)
    cols = jax.lax.broadcasted_iota(jnp.int32, (_L, _L), 1)
    s_lo = ((rows == 2 * cols) & (cols < _L // 2)).astype(jnp.float32)
    s_hi = ((rows == 2 * (cols - _L // 2)) & (cols >= _L // 2)).astype(jnp.float32)
    h = g.reshape(_R // 2, 2, _L)
    gpack = (jax.lax.dot(h[:, 0, :], s_lo, preferred_element_type=jnp.float32)
             + jax.lax.dot(h[:, 1, :], s_hi, preferred_element_type=jnp.float32))
    gamma_ref[...] = gpack.reshape(_B // 2)


def kernel(fluidPosition, fluidVelocity, fluidArea):
    n = fluidPosition.shape[0]
    flat = 2 * n
    grid = (flat + _B - 1) // _B
    pos = fluidPosition.reshape(flat)
    vel = fluidVelocity.reshape(flat)
    vel_out, gamma = pl.pallas_call(
        _vel_kernel,
        grid=(grid,),
        in_specs=[
            pl.BlockSpec((_B,), lambda i: (i,)),
            pl.BlockSpec((_B,), lambda i: (i,)),
        ],
        out_specs=[
            pl.BlockSpec((_B,), lambda i: (i,)),
            pl.BlockSpec((_B // 2,), lambda i: (i,)),
        ],
        out_shape=[
            jax.ShapeDtypeStruct((flat,), jnp.float32),
            jax.ShapeDtypeStruct((n,), jnp.float32),
        ],
    )(pos, vel)
    return vel_out.reshape(n, 2), gamma


# speedup vs baseline: 111.1431x; 93.2993x over previous
import jax
import jax.numpy as jnp
import numpy as np
from jax.experimental import pallas as pl

_INV_EM1 = float(1.0 / (np.exp(1.0) - 1.0))
_MU = 3.5
_L = 128
_K = 512


def _vel_kernel(pos_ref, vel_ref, velout_ref, gamma_ref):
    x = pos_ref[:, 0, :]
    y = pos_ref[:, 1, :]
    vx = vel_ref[:, 0, :]
    vy = vel_ref[:, 1, :]
    m = (x >= 0.0) & (x <= 0.25) & (y >= 0.0) & (y <= 1.0)
    xr = jnp.clip(x * 4.0, 0.0, 1.0)
    t = jnp.exp(jnp.log(xr) * _MU)
    g = (jnp.exp(t) - 1.0) * _INV_EM1
    g = jnp.minimum(g, 1.0)
    velout_ref[:, 0, :] = jnp.where(m, vx + g * (1.0 - vx), vx)
    velout_ref[:, 1, :] = jnp.where(m, vy * (1.0 - g), vy)
    gamma_ref[...] = g.reshape(_K * _L)


def kernel(fluidPosition, fluidVelocity, fluidArea):
    n = fluidPosition.shape[0]
    nk = n // _L
    pos = fluidPosition.reshape(nk, _L, 2).swapaxes(1, 2)
    vel = fluidVelocity.reshape(nk, _L, 2).swapaxes(1, 2)
    grid = (nk + _K - 1) // _K
    vel_out, gamma = pl.pallas_call(
        _vel_kernel,
        grid=(grid,),
        in_specs=[
            pl.BlockSpec((_K, 2, _L), lambda i: (i, 0, 0)),
            pl.BlockSpec((_K, 2, _L), lambda i: (i, 0, 0)),
        ],
        out_specs=[
            pl.BlockSpec((_K, 2, _L), lambda i: (i, 0, 0)),
            pl.BlockSpec((_K * _L,), lambda i: (i,)),
        ],
        out_shape=[
            jax.ShapeDtypeStruct((nk, 2, _L), jnp.float32),
            jax.ShapeDtypeStruct((n,), jnp.float32),
        ],
    )(pos, vel)
    vel_out = vel_out.swapaxes(1, 2).reshape(n, 2)
    return vel_out, gamma
